# baseline (speedup 1.0000x reference)
import functools

import jax
import jax.numpy as jnp
from jax import lax
from jax.experimental import pallas as pl
from jax.experimental.pallas import tpu as pltpu
from jax.experimental.pallas import tpu_sc as plsc

_RES = 1000000
_LAT = 64
_N = 1048576
_NC = 2
_NS = 16
_NW = _NC * _NS
_BW = _N // _NW
_C = 256
_G = _BW // _C
_SUB = _C // 128


def _body(u_hbm, table_hbm, out_hbm,
          u_v0, u_v1, idx_a0, idx_a1, idx_b0, idx_b1,
          rows_a0, rows_a1, rows_b0, rows_b1, rows_o0, rows_o1,
          sem_g0, sem_g1, sem_o0, sem_o1):
    wid = lax.axis_index("s") * _NC + lax.axis_index("c")
    base0 = wid * _BW
    scale = jnp.float32(_RES - 1)
    u_v = (u_v0, u_v1)
    idx_a = (idx_a0, idx_a1)
    idx_b = (idx_b0, idx_b1)
    rows_a = (rows_a0, rows_a1)
    rows_b = (rows_b0, rows_b1)
    rows_o = (rows_o0, rows_o1)
    sem_g = (sem_g0, sem_g1)
    sem_o = (sem_o0, sem_o1)

    def gather_copies(g, b):
        cs = []
        for j in range(_SUB):
            cs.append(pltpu.make_async_copy(
                table_hbm.at[idx_a[b].at[j]],
                rows_a[b].at[pl.ds(j * 128, 128)], sem_g[b]))
            cs.append(pltpu.make_async_copy(
                table_hbm.at[idx_b[b].at[j]],
                rows_b[b].at[pl.ds(j * 128, 128)], sem_g[b]))
        return cs

    def prep(g, b):
        base = base0 + g * _C
        pltpu.sync_copy(u_hbm.at[pl.ds(base, _C)], u_v[b])

        def idx_body(j, c):
            for k in range(8):
                off = j * 128 + k * 16
                u16 = u_v[b][pl.ds(off, 16)]
                f = u16 * scale
                ix = f.astype(jnp.int32)
                fl = ix.astype(jnp.float32)
                idx_a[b][j, pl.ds(k * 16, 16)] = ix
                idx_b[b][j, pl.ds(k * 16, 16)] = ix + 1
                u_v[b][pl.ds(off, 16)] = f - fl
            return c

        lax.fori_loop(0, _SUB, idx_body, 0, unroll=True)
        for c in gather_copies(g, b):
            c.start()

    def cons(g, b, first):
        base = base0 + g * _C
        for c in gather_copies(g, b):
            c.wait()
        if not first:
            pltpu.make_async_copy(
                rows_o[b], out_hbm.at[pl.ds(base, _C)], sem_o[b]).wait()

        def lerp_body(blk, c):
            i0 = blk * 16
            av = u_v[b][pl.ds(i0, 16)]
            for l in range(16):
                al = jnp.full((16,), av[l], jnp.float32)
                for r in range(4):
                    a = rows_a[b][i0 + l, pl.ds(r * 16, 16)]
                    bb = rows_b[b][i0 + l, pl.ds(r * 16, 16)]
                    rows_o[b][i0 + l, pl.ds(r * 16, 16)] = a + al * (bb - a)
            return c

        lax.fori_loop(0, _C // 16, lerp_body, 0, unroll=False)
        pltpu.make_async_copy(
            rows_o[b], out_hbm.at[pl.ds(base, _C)], sem_o[b]).start()

    prep(0, 0)
    prep(1, 1)

    cons(0, 0, True)
    prep(2, 0)
    cons(1, 1, True)
    prep(3, 1)

    def pair(gg, carry):
        for b in range(2):
            g = gg * 2 + b
            cons(g, b, False)

            @pl.when(g + 2 < _G)
            def _():
                prep(g + 2, b)
        return carry

    lax.fori_loop(1, _G // 2, pair, 0, unroll=False)

    pltpu.make_async_copy(
        rows_o[0], out_hbm.at[pl.ds(base0 + (_G - 2) * _C, _C)], sem_o[0]).wait()
    pltpu.make_async_copy(
        rows_o[1], out_hbm.at[pl.ds(base0 + (_G - 1) * _C, _C)], sem_o[1]).wait()


def kernel(uList, table):
    mesh = plsc.VectorSubcoreMesh(core_axis_name="c", subcore_axis_name="s")
    k = functools.partial(
        pl.kernel,
        mesh=mesh,
        out_type=jax.ShapeDtypeStruct((_N, _LAT), jnp.float32),
        compiler_params=pltpu.CompilerParams(use_tc_tiling_on_sc=False),
        scratch_types=[
            pltpu.VMEM((_C,), jnp.float32),
            pltpu.VMEM((_C,), jnp.float32),
            pltpu.VMEM((_SUB, 128), jnp.int32),
            pltpu.VMEM((_SUB, 128), jnp.int32),
            pltpu.VMEM((_SUB, 128), jnp.int32),
            pltpu.VMEM((_SUB, 128), jnp.int32),
            pltpu.VMEM((_C, _LAT), jnp.float32),
            pltpu.VMEM((_C, _LAT), jnp.float32),
            pltpu.VMEM((_C, _LAT), jnp.float32),
            pltpu.VMEM((_C, _LAT), jnp.float32),
            pltpu.VMEM((_C, _LAT), jnp.float32),
            pltpu.VMEM((_C, _LAT), jnp.float32),
            pltpu.SemaphoreType.DMA,
            pltpu.SemaphoreType.DMA,
            pltpu.SemaphoreType.DMA,
            pltpu.SemaphoreType.DMA,
        ],
    )(_body)
    return k(uList, table)
